# Initial kernel scaffold; baseline (speedup 1.0000x reference)
import jax
import jax.numpy as jnp
from jax.experimental import pallas as pl


def kernel(embed_0, embed_1, table_0, table_1):
    raise NotImplementedError("write your pallas kernel here")


# speedup vs baseline: 1.7531x; 1.7531x over previous
import functools

import jax
import jax.numpy as jnp
from jax import lax
from jax.experimental import pallas as pl
from jax.experimental.pallas import tpu as pltpu
from jax.experimental.pallas import tpu_sc as plsc

BATCH = 4096
HIST = 50
DIM = 64
NUM_WORKERS = 32
FLAT = BATCH * HIST
PER_WORKER = FLAT // NUM_WORKERS
CHUNK = 128
N_CHUNKS = PER_WORKER // CHUNK


def _build_sc_call():
    mesh = plsc.VectorSubcoreMesh(core_axis_name="c", subcore_axis_name="s")

    @functools.partial(
        pl.kernel,
        mesh=mesh,
        compiler_params=pltpu.CompilerParams(use_tc_tiling_on_sc=False),
        out_type=jax.ShapeDtypeStruct((BATCH * 2 * HIST, DIM), jnp.float32),
        scratch_types=[
            pltpu.VMEM((CHUNK,), jnp.int32),
            pltpu.VMEM((CHUNK,), jnp.int32),
            pltpu.VMEM((CHUNK, DIM), jnp.float32),
            pltpu.SemaphoreType.DMA,
        ],
    )
    def sc_kernel(idx0, dst0, idx1, dst1, t0, t1, out, idxv, dstv, rows, sem):
        wid = lax.axis_index("s") * 2 + lax.axis_index("c")
        base = wid * PER_WORKER

        def chunk_body(c, carry):
            off = base + c * CHUNK
            pltpu.sync_copy(idx0.at[pl.ds(off, CHUNK)], idxv)
            pltpu.sync_copy(dst0.at[pl.ds(off, CHUNK)], dstv)
            pltpu.async_copy(t0.at[idxv], rows, sem).wait()
            pltpu.async_copy(rows, out.at[dstv], sem).wait()
            pltpu.sync_copy(idx1.at[pl.ds(off, CHUNK)], idxv)
            pltpu.sync_copy(dst1.at[pl.ds(off, CHUNK)], dstv)
            pltpu.async_copy(t1.at[idxv], rows, sem).wait()
            pltpu.async_copy(rows, out.at[dstv], sem).wait()
            return carry

        lax.fori_loop(0, N_CHUNKS, chunk_body, 0)

    return sc_kernel


_sc_call = _build_sc_call()


def kernel(embed_0, embed_1, table_0, table_1):
    idx0 = embed_0.astype(jnp.int32).reshape(-1)
    idx1 = embed_1.astype(jnp.int32).reshape(-1)
    i = jnp.arange(FLAT, dtype=jnp.int32)
    dst0 = i + (i // HIST) * HIST
    dst1 = dst0 + HIST
    out = _sc_call(idx0, dst0, idx1, dst1, table_0, table_1)
    return out.reshape(BATCH, 2 * HIST * DIM)
